# baseline (speedup 1.0000x reference)
import functools

import jax
import jax.numpy as jnp
from jax import lax
from jax.experimental import pallas as pl
from jax.experimental.pallas import tpu as pltpu
from jax.experimental.pallas import tpu_sc as plsc

EMBED = 64
IMG_TOK = 151669
BATCH = 16
SEQ = 2048
TOT = BATCH * SEQ

NC, NS = 2, 16
NW = NC * NS
ROWS_PER_W = TOT // NW


@functools.cache
def _build_sc_gather():
    mesh = plsc.VectorSubcoreMesh(
        core_axis_name="c", subcore_axis_name="s", num_cores=NC, num_subcores=NS
    )

    @functools.partial(
        pl.kernel,
        mesh=mesh,
        out_type=jax.ShapeDtypeStruct((TOT, EMBED), jnp.float32),
        scratch_types=[
            pltpu.VMEM((8, 128), jnp.int32),
            pltpu.SemaphoreType.DMA,
        ],
    )
    def _sc_gather(table2d, ids2d, out_hbm, ids_v, sem):
        wid = lax.axis_index("s") * NC + lax.axis_index("c")
        base = wid * ROWS_PER_W
        pltpu.sync_copy(ids2d.at[pl.ds(wid * 8, 8)], ids_v)

        def issue(g, _):
            p = g * 16
            v = ids_v[p // 128, pl.ds(p % 128, 16)]
            for k in range(16):
                pltpu.async_copy(
                    table2d.at[pl.ds(v[k], 1)],
                    out_hbm.at[pl.ds(base + p + k, 1)],
                    sem,
                )
            return 0

        lax.fori_loop(0, ROWS_PER_W // 16, issue, 0)
        pltpu.make_async_copy(
            table2d.at[pl.ds(0, ROWS_PER_W)],
            out_hbm.at[pl.ds(base, ROWS_PER_W)],
            sem,
        ).wait()

    return _sc_gather


def _mask_body(ids_ref, attn_ref, am_out, im_out):
    am_out[...] = attn_ref[...] == 1
    im_out[...] = ids_ref[...] == IMG_TOK


def _masks_tc(input_ids, attention_mask):
    return pl.pallas_call(
        _mask_body,
        out_shape=(
            jax.ShapeDtypeStruct((BATCH, SEQ), jnp.bool_),
            jax.ShapeDtypeStruct((BATCH, SEQ), jnp.bool_),
        ),
    )(input_ids, attention_mask)


def kernel(pixel_values, input_ids, attention_mask, text_proj_weight):
    del pixel_values
    ids32 = input_ids.astype(jnp.int32)
    ids2d = ids32.reshape(TOT // 128, 128)
    flat = _build_sc_gather()(text_proj_weight, ids2d)
    hidden_states = flat.reshape(BATCH, SEQ, EMBED)
    attn_mask, image_mask = _masks_tc(ids32, attention_mask.astype(jnp.int32))
    return (hidden_states, attn_mask, image_mask)


# speedup vs baseline: 2.5545x; 2.5545x over previous
import functools

import jax
import jax.numpy as jnp
from jax import lax
from jax.experimental import pallas as pl
from jax.experimental.pallas import tpu as pltpu
from jax.experimental.pallas import tpu_sc as plsc

EMBED = 64
IMG_TOK = 151669
BATCH = 16
SEQ = 2048
TOT = BATCH * SEQ
NPAIR = 100000

NC, NS = 2, 16
NW = NC * NS
ROWS_PER_W = TOT // NW
CS = 64
NCH = ROWS_PER_W // CS
L = 16


@functools.cache
def _build_sc_gather():
    mesh = plsc.VectorSubcoreMesh(
        core_axis_name="c", subcore_axis_name="s", num_cores=NC, num_subcores=NS
    )

    @functools.partial(
        pl.kernel,
        mesh=mesh,
        out_type=jax.ShapeDtypeStruct((TOT, EMBED), jnp.float32),
        scratch_types=[
            pltpu.VMEM((8, 128), jnp.int32),
            pltpu.VMEM((8, 128), jnp.int32),
            pltpu.VMEM((8, 128), jnp.int32),
            pltpu.VMEM((CS, 128), jnp.float32),
            pltpu.VMEM((CS, 128), jnp.float32),
            pltpu.VMEM((CS, EMBED), jnp.float32),
            pltpu.VMEM((CS, EMBED), jnp.float32),
            pltpu.SemaphoreType.DMA,
            pltpu.SemaphoreType.DMA,
        ],
        compiler_params=pltpu.CompilerParams(needs_layout_passes=False),
    )
    def _sc_gather(
        table2, ids2d, out_hbm, ids_v, pidx_v, half_v, dstA, dstB, rowA, rowB,
        semA, semB,
    ):
        wid = lax.axis_index("s") * NC + lax.axis_index("c")
        pltpu.sync_copy(ids2d.at[pl.ds(wid * 8, 8)], ids_v)

        def prep(i, _):
            r, c = i // 8, (i % 8) * L
            v = ids_v[r, pl.ds(c, L)]
            pidx_v[r, pl.ds(c, L)] = lax.shift_right_logical(v, 1)
            half_v[r, pl.ds(c, L)] = lax.bitwise_and(v, 1) * EMBED
            return 0

        lax.fori_loop(0, (8 * 128) // L, prep, 0)

        def fire(g, dst, sem):
            idx_ref = pidx_v.at[(g * CS) // 128, pl.ds((g * CS) % 128, CS)]
            pltpu.async_copy(table2.at[idx_ref], dst, sem)

        def drain(dst, sem):
            pltpu.make_async_copy(table2.at[pl.ds(0, CS)], dst, sem).wait()

        iota = lax.iota(jnp.int32, L)

        def extract(g, dst, row):
            for h in range(CS // L):
                p = g * CS + h * L
                off = half_v[p // 128, pl.ds(p % 128, L)]
                r_ix = iota + (h * L)

                def col(c, _):
                    for u in range(4):
                        c_ix = jnp.zeros((L,), jnp.int32) + (c * 4 + u)
                        v = plsc.load_gather(dst, [r_ix, off + c_ix])
                        plsc.store_scatter(row, [r_ix, c_ix], v)
                    return 0

                lax.fori_loop(0, EMBED // 4, col, 0)

        def flush(g, row):
            pltpu.sync_copy(
                row, out_hbm.at[pl.ds(wid * ROWS_PER_W + g * CS, CS)]
            )

        fire(0, dstA, semA)

        def step(h, _):
            g0 = 2 * h
            drain(dstA, semA)
            fire(g0 + 1, dstB, semB)
            extract(g0, dstA, rowA)
            flush(g0, rowA)

            @pl.when(g0 + 2 < NCH)
            def _():
                fire(g0 + 2, dstA, semA)

            drain(dstB, semB)
            extract(g0 + 1, dstB, rowB)
            flush(g0 + 1, rowB)
            return 0

        lax.fori_loop(0, NCH // 2, step, 0)

    return _sc_gather


def _mask_body(ids_ref, attn_ref, am_out, im_out):
    am_out[...] = attn_ref[...] == 1
    im_out[...] = ids_ref[...] == IMG_TOK


def _masks_tc(input_ids, attention_mask):
    return pl.pallas_call(
        _mask_body,
        out_shape=(
            jax.ShapeDtypeStruct((BATCH, SEQ), jnp.bool_),
            jax.ShapeDtypeStruct((BATCH, SEQ), jnp.bool_),
        ),
    )(input_ids, attention_mask)


def kernel(pixel_values, input_ids, attention_mask, text_proj_weight):
    del pixel_values
    ids32 = input_ids.astype(jnp.int32)
    ids2d = ids32.reshape(TOT // 128, 128)
    table2 = text_proj_weight.reshape(NPAIR, 2 * EMBED)
    flat = _build_sc_gather()(table2, ids2d)
    hidden_states = flat.reshape(BATCH, SEQ, EMBED)
    attn_mask, image_mask = _masks_tc(ids32, attention_mask.astype(jnp.int32))
    return (hidden_states, attn_mask, image_mask)
